# Initial kernel scaffold; baseline (speedup 1.0000x reference)
import jax
import jax.numpy as jnp
from jax.experimental import pallas as pl


def kernel(x, adj, rep_adj_dis, W_gin, W_v):
    raise NotImplementedError("write your pallas kernel here")


# speedup vs baseline: 22.1736x; 22.1736x over previous
import jax
import jax.numpy as jnp
from jax.experimental import pallas as pl

_B = 2
_T = 196
_DIM = 768
_HEADS = 12
_DH = _DIM // _HEADS
_TOPK = 49


def _gelu_exact(x):
    return 0.5 * x * (1.0 + jax.lax.erf(x * (2.0 ** -0.5)))


def _head_kernel(adj_ref, xh_ref, wq_ref, wk_ref, wv_ref, wo_ref, out_ref):
    h = pl.program_id(1)
    adj = adj_ref[0, 0]
    xh = xh_ref[0, 0]
    hi = jax.lax.Precision.HIGHEST

    msg = jnp.dot(adj, xh, precision=hi, preferred_element_type=jnp.float32) + xh
    q = jnp.dot(msg, wq_ref[...], precision=hi, preferred_element_type=jnp.float32)
    k = jnp.dot(msg, wk_ref[...], precision=hi, preferred_element_type=jnp.float32)
    v = jnp.dot(msg, wv_ref[...], precision=hi, preferred_element_type=jnp.float32)

    min32 = jnp.int32(-(2 ** 31))
    bits = jax.lax.bitcast_convert_type(adj, jnp.int32)
    s = bits ^ (jax.lax.shift_right_arithmetic(bits, 31) & jnp.int32(0x7FFFFFFF))

    def body(i, t_u):
        t_try = t_u | jnp.left_shift(jnp.int32(1), 31 - i)
        thr_s = t_try ^ min32
        cnt = jnp.sum((s >= thr_s).astype(jnp.int32), axis=1, keepdims=True)
        return jnp.where(cnt >= _TOPK, t_try, t_u)

    t_u = jax.lax.fori_loop(0, 32, body, jnp.zeros((_T, 1), jnp.int32))
    thr_s = t_u ^ min32

    gt = s > thr_s
    eq = s == thr_s
    cnt_gt = jnp.sum(gt.astype(jnp.int32), axis=1, keepdims=True)
    rows = jax.lax.broadcasted_iota(jnp.int32, (_T, _T), 0)
    cols = jax.lax.broadcasted_iota(jnp.int32, (_T, _T), 1)
    upper = (rows < cols).astype(jnp.float32)
    excl = jnp.dot(eq.astype(jnp.float32), upper, precision=hi,
                   preferred_element_type=jnp.float32)
    keep_eq = eq & (excl.astype(jnp.int32) < (_TOPK - cnt_gt))
    mask = gt | keep_eq

    scale = _DIM ** -0.5
    scores = jax.lax.dot_general(q * scale, k, (((1,), (1,)), ((), ())),
                                 precision=hi, preferred_element_type=jnp.float32)
    scores = jnp.where(mask, scores, -1e30)
    m = jnp.max(scores, axis=1, keepdims=True)
    e = jnp.exp(scores - m)
    p = e / jnp.sum(e, axis=1, keepdims=True)
    o = jnp.dot(p, v, precision=hi, preferred_element_type=jnp.float32)

    contrib = jnp.dot(_gelu_exact(o), wo_ref[0], precision=hi,
                      preferred_element_type=jnp.float32)

    @pl.when(h == 0)
    def _():
        out_ref[0] = contrib

    @pl.when(h != 0)
    def _():
        out_ref[0] = out_ref[0] + contrib


def kernel(x, adj, rep_adj_dis, W_gin, W_v):
    del rep_adj_dis
    xh = x.reshape(_B, _T, _HEADS, _DH).transpose(0, 2, 1, 3)
    adj4 = adj.reshape(_B, _HEADS, _T, _T)
    wq = W_gin[:, 0 * _DH:1 * _DH]
    wk = W_gin[:, 1 * _DH:2 * _DH]
    wv = W_gin[:, 2 * _DH:3 * _DH]
    wo = W_v.reshape(_HEADS, _DH, _DIM)

    out = pl.pallas_call(
        _head_kernel,
        grid=(_B, _HEADS),
        in_specs=[
            pl.BlockSpec((1, 1, _T, _T), lambda b, h: (b, h, 0, 0)),
            pl.BlockSpec((1, 1, _T, _DH), lambda b, h: (b, h, 0, 0)),
            pl.BlockSpec((_DH, _DH), lambda b, h: (0, 0)),
            pl.BlockSpec((_DH, _DH), lambda b, h: (0, 0)),
            pl.BlockSpec((_DH, _DH), lambda b, h: (0, 0)),
            pl.BlockSpec((1, _DH, _DIM), lambda b, h: (h, 0, 0)),
        ],
        out_specs=pl.BlockSpec((1, _T, _DIM), lambda b, h: (b, 0, 0)),
        out_shape=jax.ShapeDtypeStruct((_B, _T, _DIM), jnp.float32),
    )(adj4, xh, wq, wk, wv, wo)
    return out


# speedup vs baseline: 36.2470x; 1.6347x over previous
import jax
import jax.numpy as jnp
from jax.experimental import pallas as pl

_B = 2
_T = 196
_DIM = 768
_HEADS = 12
_DH = _DIM // _HEADS
_TOPK = 49


def _gelu_exact(x):
    return 0.5 * x * (1.0 + jax.lax.erf(x * (2.0 ** -0.5)))


def _head_kernel(adjT_ref, xh_ref, wq_ref, wk_ref, wv_ref, wo_ref, out_ref):
    h = pl.program_id(1)
    adjT = adjT_ref[0, 0]
    xh = xh_ref[0, 0]
    hi = jax.lax.Precision.HIGHEST
    tn = (((0,), (0,)), ((), ()))
    nt = (((1,), (1,)), ((), ()))

    msg = jax.lax.dot_general(adjT, xh, tn, precision=hi,
                              preferred_element_type=jnp.float32) + xh
    q = jnp.dot(msg, wq_ref[...], precision=hi, preferred_element_type=jnp.float32)
    k = jnp.dot(msg, wk_ref[...], precision=hi, preferred_element_type=jnp.float32)
    v = jnp.dot(msg, wv_ref[...], precision=hi, preferred_element_type=jnp.float32)

    min32 = jnp.int32(-(2 ** 31))
    bits = jax.lax.bitcast_convert_type(adjT, jnp.int32)
    sT = bits ^ (jax.lax.shift_right_arithmetic(bits, 31) & jnp.int32(0x7FFFFFFF))

    def body(i, t_u):
        t_try = t_u | jnp.left_shift(jnp.int32(1), 31 - i)
        thr_s = t_try ^ min32
        ge = jnp.where(sT >= thr_s, 1.0, 0.0)
        cnt = jnp.sum(ge, axis=0, keepdims=True)
        return jnp.where(cnt >= _TOPK, t_try, t_u)

    t_u = jax.lax.fori_loop(0, 32, body, jnp.zeros((1, _T), jnp.int32))
    thr_s = t_u ^ min32

    gtT = sT > thr_s
    eqT = sT == thr_s
    cnt_gt = jnp.sum(jnp.where(gtT, 1.0, 0.0), axis=0, keepdims=True)
    rows = jax.lax.broadcasted_iota(jnp.int32, (_T, _T), 0)
    cols = jax.lax.broadcasted_iota(jnp.int32, (_T, _T), 1)
    upper = (rows < cols).astype(jnp.float32)
    exclT = jax.lax.dot_general(upper, eqT.astype(jnp.float32), tn,
                                preferred_element_type=jnp.float32)
    keepT = eqT & (exclT < (_TOPK - cnt_gt))
    maskT = gtT | keepT

    scale = _DIM ** -0.5
    scoresT = jax.lax.dot_general(k, q * scale, nt, precision=hi,
                                  preferred_element_type=jnp.float32)
    scoresT = jnp.where(maskT, scoresT, -1e30)
    m = jnp.max(scoresT, axis=0, keepdims=True)
    e = jnp.exp(scoresT - m)
    p = e / jnp.sum(e, axis=0, keepdims=True)
    o = jax.lax.dot_general(p, v, tn, precision=hi,
                            preferred_element_type=jnp.float32)

    contrib = jnp.dot(_gelu_exact(o), wo_ref[0], precision=hi,
                      preferred_element_type=jnp.float32)

    @pl.when(h == 0)
    def _():
        out_ref[0] = contrib

    @pl.when(h != 0)
    def _():
        out_ref[0] = out_ref[0] + contrib


def kernel(x, adj, rep_adj_dis, W_gin, W_v):
    del rep_adj_dis
    xh = x.reshape(_B, _T, _HEADS, _DH).transpose(0, 2, 1, 3)
    adjT4 = adj.reshape(_B, _HEADS, _T, _T).transpose(0, 1, 3, 2)
    wq = W_gin[:, 0 * _DH:1 * _DH]
    wk = W_gin[:, 1 * _DH:2 * _DH]
    wv = W_gin[:, 2 * _DH:3 * _DH]
    wo = W_v.reshape(_HEADS, _DH, _DIM)

    out = pl.pallas_call(
        _head_kernel,
        grid=(_B, _HEADS),
        in_specs=[
            pl.BlockSpec((1, 1, _T, _T), lambda b, h: (b, h, 0, 0)),
            pl.BlockSpec((1, 1, _T, _DH), lambda b, h: (b, h, 0, 0)),
            pl.BlockSpec((_DH, _DH), lambda b, h: (0, 0)),
            pl.BlockSpec((_DH, _DH), lambda b, h: (0, 0)),
            pl.BlockSpec((_DH, _DH), lambda b, h: (0, 0)),
            pl.BlockSpec((1, _DH, _DIM), lambda b, h: (h, 0, 0)),
        ],
        out_specs=pl.BlockSpec((1, _T, _DIM), lambda b, h: (b, 0, 0)),
        out_shape=jax.ShapeDtypeStruct((_B, _T, _DIM), jnp.float32),
    )(adjT4, xh, wq, wk, wv, wo)
    return out


# speedup vs baseline: 48.5511x; 1.3395x over previous
import jax
import jax.numpy as jnp
from jax.experimental import pallas as pl

_B = 2
_T = 196
_DIM = 768
_HEADS = 12
_DH = _DIM // _HEADS
_TOPK = 49


def _gelu_exact(x):
    return 0.5 * x * (1.0 + jax.lax.erf(x * (2.0 ** -0.5)))


def _split_bf16(a):
    ah = a.astype(jnp.bfloat16)
    al = (a - ah.astype(jnp.float32)).astype(jnp.bfloat16)
    return ah, al


def _dot3(a, b, dims):
    ah, al = _split_bf16(a)
    bh, bl = _split_bf16(b)

    def d(u, w):
        return jax.lax.dot_general(u, w, dims, preferred_element_type=jnp.float32)

    return d(ah, bh) + d(ah, bl) + d(al, bh)


def _head_kernel(adjT_ref, xh_ref, wq_ref, wk_ref, wv_ref, wo_ref, out_ref):
    h = pl.program_id(1)
    adjT = adjT_ref[0, 0]
    xh = xh_ref[0, 0]
    tn = (((0,), (0,)), ((), ()))
    nn = (((1,), (0,)), ((), ()))
    nt = (((1,), (1,)), ((), ()))

    msg = _dot3(adjT, xh, tn) + xh
    q = _dot3(msg, wq_ref[...], nn)
    k = _dot3(msg, wk_ref[...], nn)
    v = _dot3(msg, wv_ref[...], nn)

    min32 = jnp.int32(-(2 ** 31))
    bits = jax.lax.bitcast_convert_type(adjT, jnp.int32)
    sT = bits ^ (jax.lax.shift_right_arithmetic(bits, 31) & jnp.int32(0x7FFFFFFF))

    t_u = jnp.zeros((1, _T), jnp.int32)
    for i in range(32):
        t_try = t_u | jnp.int32((1 << (31 - i)) - (2 ** 32 if i == 0 else 0))
        thr = t_try ^ min32
        ge = jnp.where(sT >= thr, 1.0, 0.0)
        cnt = jnp.sum(ge, axis=0, keepdims=True)
        t_u = jnp.where(cnt >= _TOPK, t_try, t_u)
    thr_s = t_u ^ min32

    gtT = sT > thr_s
    eqT = sT == thr_s
    cnt_gt = jnp.sum(jnp.where(gtT, 1.0, 0.0), axis=0, keepdims=True)
    rows = jax.lax.broadcasted_iota(jnp.int32, (_T, _T), 0)
    cols = jax.lax.broadcasted_iota(jnp.int32, (_T, _T), 1)
    upper = (rows < cols).astype(jnp.float32)
    exclT = jax.lax.dot_general(upper, eqT.astype(jnp.float32), tn,
                                preferred_element_type=jnp.float32)
    keepT = eqT & (exclT < (_TOPK - cnt_gt))
    maskT = gtT | keepT

    scale = _DIM ** -0.5
    scoresT = _dot3(k, q * scale, nt)
    scoresT = jnp.where(maskT, scoresT, -1e30)
    m = jnp.max(scoresT, axis=0, keepdims=True)
    e = jnp.exp(scoresT - m)
    p = e / jnp.sum(e, axis=0, keepdims=True)
    o = _dot3(p, v, tn)

    contrib = _dot3(_gelu_exact(o), wo_ref[0], nn)

    @pl.when(h == 0)
    def _():
        out_ref[0] = contrib

    @pl.when(h != 0)
    def _():
        out_ref[0] = out_ref[0] + contrib


def kernel(x, adj, rep_adj_dis, W_gin, W_v):
    del rep_adj_dis
    xh = x.reshape(_B, _T, _HEADS, _DH).transpose(0, 2, 1, 3)
    adjT4 = adj.reshape(_B, _HEADS, _T, _T).transpose(0, 1, 3, 2)
    wq = W_gin[:, 0 * _DH:1 * _DH]
    wk = W_gin[:, 1 * _DH:2 * _DH]
    wv = W_gin[:, 2 * _DH:3 * _DH]
    wo = W_v.reshape(_HEADS, _DH, _DIM)

    out = pl.pallas_call(
        _head_kernel,
        grid=(_B, _HEADS),
        in_specs=[
            pl.BlockSpec((1, 1, _T, _T), lambda b, h: (b, h, 0, 0)),
            pl.BlockSpec((1, 1, _T, _DH), lambda b, h: (b, h, 0, 0)),
            pl.BlockSpec((_DH, _DH), lambda b, h: (0, 0)),
            pl.BlockSpec((_DH, _DH), lambda b, h: (0, 0)),
            pl.BlockSpec((_DH, _DH), lambda b, h: (0, 0)),
            pl.BlockSpec((1, _DH, _DIM), lambda b, h: (h, 0, 0)),
        ],
        out_specs=pl.BlockSpec((1, _T, _DIM), lambda b, h: (b, 0, 0)),
        out_shape=jax.ShapeDtypeStruct((_B, _T, _DIM), jnp.float32),
    )(adjT4, xh, wq, wk, wv, wo)
    return out


# speedup vs baseline: 61.5731x; 1.2682x over previous
import jax
import jax.numpy as jnp
from jax.experimental import pallas as pl

_B = 2
_T = 196
_DIM = 768
_HEADS = 12
_DH = _DIM // _HEADS
_TOPK = 49


def _gelu_exact(x):
    return 0.5 * x * (1.0 + jax.lax.erf(x * (2.0 ** -0.5)))


def _split_bf16(a):
    ah = a.astype(jnp.bfloat16)
    al = (a - ah.astype(jnp.float32)).astype(jnp.bfloat16)
    return ah, al


def _dot3(a, b, dims):
    ah, al = _split_bf16(a)
    bh, bl = _split_bf16(b)

    def d(u, w):
        return jax.lax.dot_general(u, w, dims, preferred_element_type=jnp.float32)

    return d(ah, bh) + d(ah, bl) + d(al, bh)

_TN = (((0,), (0,)), ((), ()))
_NN = (((1,), (0,)), ((), ()))
_NT = (((1,), (1,)), ((), ()))


def _head_contrib(adjT, xh, wq, wk, wv, wo, upper_bf):
    msg = _dot3(adjT, xh, _TN) + xh
    q = _dot3(msg, wq, _NN)
    k = _dot3(msg, wk, _NN)
    v = _dot3(msg, wv, _NN)

    min32 = jnp.int32(-(2 ** 31))
    bits = jax.lax.bitcast_convert_type(adjT, jnp.int32)
    sT = bits ^ (jax.lax.shift_right_arithmetic(bits, 31) & jnp.int32(0x7FFFFFFF))

    t_u = jnp.zeros((1, _T), jnp.int32)
    for i in range(32):
        t_try = t_u | jnp.int32((1 << (31 - i)) - (2 ** 32 if i == 0 else 0))
        thr = t_try ^ min32
        ge = jnp.where(sT >= thr, 1.0, 0.0)
        cnt = jnp.sum(ge, axis=0, keepdims=True)
        t_u = jnp.where(cnt >= _TOPK, t_try, t_u)
    thr_s = t_u ^ min32

    gtT = sT > thr_s
    eqT = sT == thr_s
    cnt_gt = jnp.sum(jnp.where(gtT, 1.0, 0.0), axis=0, keepdims=True)
    exclT = jax.lax.dot_general(upper_bf, eqT.astype(jnp.bfloat16), _TN,
                                preferred_element_type=jnp.float32)
    keepT = eqT & (exclT < (_TOPK - cnt_gt))
    maskT = gtT | keepT

    scale = _DIM ** -0.5
    scoresT = _dot3(k, q * scale, _NT)
    scoresT = jnp.where(maskT, scoresT, -1e30)
    m = jnp.max(scoresT, axis=0, keepdims=True)
    e = jnp.exp(scoresT - m)
    p = e * (1.0 / jnp.sum(e, axis=0, keepdims=True))
    o = _dot3(p, v, _TN)

    return _dot3(_gelu_exact(o), wo, _NN)


def _batch_kernel(adjT_ref, xh_ref, wq_ref, wk_ref, wv_ref, wo_ref, out_ref):
    rows = jax.lax.broadcasted_iota(jnp.int32, (_T, _T), 0)
    cols = jax.lax.broadcasted_iota(jnp.int32, (_T, _T), 1)
    upper_bf = (rows < cols).astype(jnp.bfloat16)

    wq = wq_ref[...]
    wk = wk_ref[...]
    wv = wv_ref[...]

    acc = None
    for h in range(_HEADS):
        contrib = _head_contrib(adjT_ref[0, h], xh_ref[0, h], wq, wk, wv,
                                wo_ref[h], upper_bf)
        acc = contrib if acc is None else acc + contrib
    out_ref[0] = acc


def kernel(x, adj, rep_adj_dis, W_gin, W_v):
    del rep_adj_dis
    xh = x.reshape(_B, _T, _HEADS, _DH).transpose(0, 2, 1, 3)
    adjT4 = adj.reshape(_B, _HEADS, _T, _T).transpose(0, 1, 3, 2)
    wq = W_gin[:, 0 * _DH:1 * _DH]
    wk = W_gin[:, 1 * _DH:2 * _DH]
    wv = W_gin[:, 2 * _DH:3 * _DH]
    wo = W_v.reshape(_HEADS, _DH, _DIM)

    out = pl.pallas_call(
        _batch_kernel,
        grid=(_B,),
        in_specs=[
            pl.BlockSpec((1, _HEADS, _T, _T), lambda b: (b, 0, 0, 0)),
            pl.BlockSpec((1, _HEADS, _T, _DH), lambda b: (b, 0, 0, 0)),
            pl.BlockSpec((_DH, _DH), lambda b: (0, 0)),
            pl.BlockSpec((_DH, _DH), lambda b: (0, 0)),
            pl.BlockSpec((_DH, _DH), lambda b: (0, 0)),
            pl.BlockSpec((_HEADS, _DH, _DIM), lambda b: (0, 0, 0)),
        ],
        out_specs=pl.BlockSpec((1, _T, _DIM), lambda b: (b, 0, 0)),
        out_shape=jax.ShapeDtypeStruct((_B, _T, _DIM), jnp.float32),
    )(adjT4, xh, wq, wk, wv, wo)
    return out


# speedup vs baseline: 65.4438x; 1.0629x over previous
import jax
import jax.numpy as jnp
from jax.experimental import pallas as pl

_B = 2
_T = 196
_DIM = 768
_HEADS = 12
_DH = _DIM // _HEADS
_TOPK = 49


def _gelu_exact(x):
    return 0.5 * x * (1.0 + jax.lax.erf(x * (2.0 ** -0.5)))


def _split_bf16(a):
    ah = a.astype(jnp.bfloat16)
    al = (a - ah.astype(jnp.float32)).astype(jnp.bfloat16)
    return ah, al


def _dot3(a, b, dims):
    ah, al = _split_bf16(a)
    bh, bl = _split_bf16(b)

    def d(u, w):
        return jax.lax.dot_general(u, w, dims, preferred_element_type=jnp.float32)

    return d(ah, bh) + d(ah, bl) + d(al, bh)

_TN = (((0,), (0,)), ((), ()))
_NN = (((1,), (0,)), ((), ()))
_NT = (((1,), (1,)), ((), ()))


def _head_contrib(adj, xh, wg, wo, upper_bf):
    msg = _dot3(adj, xh, _NN) + xh
    qkv = _dot3(msg, wg, _NN)
    q = qkv[:, 0 * _DH:1 * _DH]
    k = qkv[:, 1 * _DH:2 * _DH]
    v = qkv[:, 2 * _DH:3 * _DH]

    min32 = jnp.int32(-(2 ** 31))
    bits = jax.lax.bitcast_convert_type(adj.T, jnp.int32)
    sT = bits ^ (jax.lax.shift_right_arithmetic(bits, 31) & jnp.int32(0x7FFFFFFF))

    t_u = jnp.zeros((1, _T), jnp.int32)
    for i in range(32):
        t_try = t_u | jnp.int32((1 << (31 - i)) - (2 ** 32 if i == 0 else 0))
        thr = t_try ^ min32
        ge = jnp.where(sT >= thr, 1.0, 0.0)
        cnt = jnp.sum(ge, axis=0, keepdims=True)
        t_u = jnp.where(cnt >= _TOPK, t_try, t_u)
    thr_s = t_u ^ min32

    gtT = sT > thr_s
    eqT = sT == thr_s
    cnt_gt = jnp.sum(jnp.where(gtT, 1.0, 0.0), axis=0, keepdims=True)
    exclT = jax.lax.dot_general(upper_bf, eqT.astype(jnp.bfloat16), _TN,
                                preferred_element_type=jnp.float32)
    keepT = eqT & (exclT < (_TOPK - cnt_gt))
    maskT = gtT | keepT

    scale = _DIM ** -0.5
    scoresT = _dot3(k, q * scale, _NT)
    scoresT = jnp.where(maskT, scoresT, -1e30)
    m = jnp.max(scoresT, axis=0, keepdims=True)
    e = jnp.exp(scoresT - m)
    p = e * (1.0 / jnp.sum(e, axis=0, keepdims=True))
    o = _dot3(p, v, _TN)

    return _dot3(_gelu_exact(o), wo, _NN)


def _batch_kernel(adj_ref, x_ref, wg_ref, wv_ref, out_ref):
    rows = jax.lax.broadcasted_iota(jnp.int32, (_T, _T), 0)
    cols = jax.lax.broadcasted_iota(jnp.int32, (_T, _T), 1)
    upper_bf = (rows < cols).astype(jnp.bfloat16)

    wg = wg_ref[...]
    x = x_ref[0]

    acc = None
    for h in range(_HEADS):
        contrib = _head_contrib(adj_ref[h], x[:, h * _DH:(h + 1) * _DH], wg,
                                wv_ref[h * _DH:(h + 1) * _DH, :], upper_bf)
        acc = contrib if acc is None else acc + contrib
    out_ref[0] = acc


def kernel(x, adj, rep_adj_dis, W_gin, W_v):
    del rep_adj_dis

    out = pl.pallas_call(
        _batch_kernel,
        grid=(_B,),
        in_specs=[
            pl.BlockSpec((_HEADS, _T, _T), lambda b: (b, 0, 0)),
            pl.BlockSpec((1, _T, _DIM), lambda b: (b, 0, 0)),
            pl.BlockSpec((_DH, 3 * _DH), lambda b: (0, 0)),
            pl.BlockSpec((_DIM, _DIM), lambda b: (0, 0)),
        ],
        out_specs=pl.BlockSpec((1, _T, _DIM), lambda b: (b, 0, 0)),
        out_shape=jax.ShapeDtypeStruct((_B, _T, _DIM), jnp.float32),
    )(adj, x, W_gin, W_v)
    return out


# speedup vs baseline: 65.5944x; 1.0023x over previous
import jax
import jax.numpy as jnp
from jax.experimental import pallas as pl
from jax.experimental.pallas import tpu as pltpu

_B = 2
_T = 196
_DIM = 768
_HEADS = 12
_DH = _DIM // _HEADS
_TOPK = 49


def _gelu_exact(x):
    return 0.5 * x * (1.0 + jax.lax.erf(x * (2.0 ** -0.5)))


def _split_bf16(a):
    ah = a.astype(jnp.bfloat16)
    al = (a - ah.astype(jnp.float32)).astype(jnp.bfloat16)
    return ah, al


def _dot3(a, b, dims):
    ah, al = _split_bf16(a)
    bh, bl = _split_bf16(b)

    def d(u, w):
        return jax.lax.dot_general(u, w, dims, preferred_element_type=jnp.float32)

    return d(ah, bh) + d(ah, bl) + d(al, bh)

_TN = (((0,), (0,)), ((), ()))
_NN = (((1,), (0,)), ((), ()))
_NT = (((1,), (1,)), ((), ()))


def _head_contrib(adj, xh, wg, wo, upper_bf):
    msg = _dot3(adj, xh, _NN) + xh
    qkv = _dot3(msg, wg, _NN)
    q = qkv[:, 0 * _DH:1 * _DH]
    k = qkv[:, 1 * _DH:2 * _DH]
    v = qkv[:, 2 * _DH:3 * _DH]

    min32 = jnp.int32(-(2 ** 31))
    bits = jax.lax.bitcast_convert_type(adj.T, jnp.int32)
    sT = bits ^ (jax.lax.shift_right_arithmetic(bits, 31) & jnp.int32(0x7FFFFFFF))

    t_u = jnp.zeros((1, _T), jnp.int32)
    for i in range(32):
        t_try = t_u | jnp.int32((1 << (31 - i)) - (2 ** 32 if i == 0 else 0))
        thr = t_try ^ min32
        ge = jnp.where(sT >= thr, 1.0, 0.0)
        cnt = jnp.sum(ge, axis=0, keepdims=True)
        t_u = jnp.where(cnt >= _TOPK, t_try, t_u)
    thr_s = t_u ^ min32

    gtT = sT > thr_s
    eqT = sT == thr_s
    cnt_gt = jnp.sum(jnp.where(gtT, 1.0, 0.0), axis=0, keepdims=True)
    exclT = jax.lax.dot_general(upper_bf, eqT.astype(jnp.bfloat16), _TN,
                                preferred_element_type=jnp.float32)
    keepT = eqT & (exclT < (_TOPK - cnt_gt))
    maskT = gtT | keepT

    scale = _DIM ** -0.5
    scoresT = _dot3(k, q * scale, _NT)
    scoresT = jnp.where(maskT, scoresT, -1e30)
    m = jnp.max(scoresT, axis=0, keepdims=True)
    e = jnp.exp(scoresT - m)
    p = e * (1.0 / jnp.sum(e, axis=0, keepdims=True))
    o = _dot3(p, v, _TN)

    return _dot3(_gelu_exact(o), wo, _NN)


def _batch_kernel(adj_ref, x_ref, wg_ref, wv_ref, out_ref):
    rows = jax.lax.broadcasted_iota(jnp.int32, (_T, _T), 0)
    cols = jax.lax.broadcasted_iota(jnp.int32, (_T, _T), 1)
    upper_bf = (rows < cols).astype(jnp.bfloat16)

    wg = wg_ref[...]
    x = x_ref[0]

    acc = None
    for h in range(_HEADS):
        contrib = _head_contrib(adj_ref[h], x[:, h * _DH:(h + 1) * _DH], wg,
                                wv_ref[h * _DH:(h + 1) * _DH, :], upper_bf)
        acc = contrib if acc is None else acc + contrib
    out_ref[0] = acc


def kernel(x, adj, rep_adj_dis, W_gin, W_v):
    del rep_adj_dis

    out = pl.pallas_call(
        _batch_kernel,
        grid=(_B,),
        in_specs=[
            pl.BlockSpec((_HEADS, _T, _T), lambda b: (b, 0, 0)),
            pl.BlockSpec((1, _T, _DIM), lambda b: (b, 0, 0)),
            pl.BlockSpec((_DH, 3 * _DH), lambda b: (0, 0)),
            pl.BlockSpec((_DIM, _DIM), lambda b: (0, 0)),
        ],
        out_specs=pl.BlockSpec((1, _T, _DIM), lambda b: (b, 0, 0)),
        out_shape=jax.ShapeDtypeStruct((_B, _T, _DIM), jnp.float32),
        compiler_params=pltpu.CompilerParams(
            dimension_semantics=("parallel",)),
    )(adj, x, W_gin, W_v)
    return out
